# baseline (speedup 1.0000x reference)
import functools

import jax
import jax.numpy as jnp
from jax import lax
from jax.experimental import pallas as pl
from jax.experimental.pallas import tpu as pltpu
from jax.experimental.pallas import tpu_sc as plsc

_NUM_CODES = 1024
_BASIS_DIM = 2700
_BETA = 0.25
_BLK = 256


_PDIM = (_BASIS_DIM + 127) // 128 * 128


def _vq_tc_body(slots_ref, w_ref, b_ref, basis_ref,
                idx_ref, idx2_ref, loss_ref, ent_ref, tp_ref,
                acc_ref, cn_ref, msum_ref):
    i = pl.program_id(0)
    nblk = pl.num_programs(0)
    trows = _NUM_CODES // nblk

    @pl.when(i == 0)
    def _init():
        bsq = basis_ref[...] * basis_ref[...]
        cn_ref[...] = jnp.sum(bsq, axis=1)[None, :]
        acc_ref[...] = jnp.zeros_like(acc_ref)
        msum_ref[0] = 0.0

    tp_ref[0, :, :_BASIS_DIM] = basis_ref[pl.ds(i * trows, trows), :]
    tp_ref[0, :, _BASIS_DIM:] = jnp.zeros((trows, _PDIM - _BASIS_DIM),
                                          jnp.float32)

    z = jnp.dot(slots_ref[...], w_ref[...],
                preferred_element_type=jnp.float32) + b_ref[...]
    g = lax.dot_general(z, basis_ref[...], (((1,), (1,)), ((), ())),
                        preferred_element_type=jnp.float32)
    zn = jnp.sum(z * z, axis=1, keepdims=True)
    dist = zn - 2.0 * g + cn_ref[...]
    m = jnp.min(dist, axis=1, keepdims=True)
    idx = jnp.argmin(dist, axis=1).astype(jnp.int32)
    idx_ref[...] = idx
    kk = idx2_ref.shape[1]
    for r in range(_BLK // kk):
        idx2_ref[r, :] = idx[r * kk:(r + 1) * kk]
    p = jnp.exp(m - dist)
    p = p / jnp.sum(p, axis=1, keepdims=True)
    acc_ref[...] += jnp.sum(p, axis=0, keepdims=True)
    msum_ref[0] += jnp.sum(m)

    @pl.when(i == nblk - 1)
    def _fin():
        nrows = nblk * _BLK
        avg = acc_ref[...] / nrows
        ent_ref[0, 0] = -jnp.sum(avg * jnp.log(avg + 1e-8))
        loss_ref[0, 0] = (_BETA / (nrows * _BASIS_DIM)) * msum_ref[0]


def _vq_tc(slots2d, w, b2, basis, bsz, k):
    nrows, d = slots2d.shape
    nblk = nrows // _BLK
    rpb = _BLK // k
    return pl.pallas_call(
        _vq_tc_body,
        grid=(nblk,),
        in_specs=[
            pl.BlockSpec((_BLK, d), lambda i: (i, 0)),
            pl.BlockSpec(w.shape, lambda i: (0, 0)),
            pl.BlockSpec(b2.shape, lambda i: (0, 0)),
            pl.BlockSpec(basis.shape, lambda i: (0, 0)),
        ],
        out_specs=[
            pl.BlockSpec((_BLK,), lambda i: (i,)),
            pl.BlockSpec((rpb, k), lambda i: (i, 0)),
            pl.BlockSpec(memory_space=pltpu.SMEM),
            pl.BlockSpec(memory_space=pltpu.SMEM),
            pl.BlockSpec((1, _NUM_CODES // nblk, _PDIM), lambda i: (i, 0, 0)),
        ],
        out_shape=[
            jax.ShapeDtypeStruct((nblk * _BLK,), jnp.int32),
            jax.ShapeDtypeStruct((bsz, k), jnp.int32),
            jax.ShapeDtypeStruct((1, 1), jnp.float32),
            jax.ShapeDtypeStruct((1, 1), jnp.float32),
            jax.ShapeDtypeStruct((nblk, _NUM_CODES // nblk, _PDIM),
                                 jnp.float32),
        ],
        scratch_shapes=[
            pltpu.VMEM((1, _NUM_CODES), jnp.float32),
            pltpu.VMEM((1, _NUM_CODES), jnp.float32),
            pltpu.SMEM((1,), jnp.float32),
        ],
    )(slots2d, w, b2, basis)


def _sc_gather_call(table_pad, idx_flat, dim):
    nrows = idx_flat.shape[0]
    pdim = table_pad.shape[1]
    info = plsc.get_sparse_core_info()
    ncores = info.num_cores
    nw = ncores * info.num_subcores
    rpw = nrows // nw
    chunk = 8
    nbuf = 4
    nch = rpw // chunk

    @functools.partial(
        pl.kernel,
        out_type=jax.ShapeDtypeStruct((nrows, dim), jnp.float32),
        mesh=plsc.VectorSubcoreMesh(core_axis_name="c", subcore_axis_name="s"),
        scratch_types=(
            [pltpu.VMEM((rpw,), jnp.int32)]
            + [pltpu.VMEM((chunk, pdim), jnp.float32)] * nbuf
            + [pltpu.SemaphoreType.DMA] * (2 * nbuf)
        ),
    )
    def gk(table_hbm, idx_hbm, out_hbm, idx_v, *bufsem):
        bufs = bufsem[:nbuf]
        gs = bufsem[nbuf:2 * nbuf]
        ws = bufsem[2 * nbuf:]
        wid = lax.axis_index("s") * ncores + lax.axis_index("c")
        base = wid * rpw
        pltpu.sync_copy(idx_hbm.at[pl.ds(base, rpw)], idx_v)
        gh = {}
        wh = {}
        for c in range(min(nbuf, nch)):
            gh[c] = pltpu.async_copy(
                table_hbm.at[idx_v.at[pl.ds(c * chunk, chunk)]],
                bufs[c % nbuf], gs[c % nbuf])
        for c in range(nch):
            nx = c + 2
            if nbuf <= nx < nch:
                wh[nx - nbuf].wait()
                gh[nx] = pltpu.async_copy(
                    table_hbm.at[idx_v.at[pl.ds(nx * chunk, chunk)]],
                    bufs[nx % nbuf], gs[nx % nbuf])
            gh[c].wait()
            wh[c] = pltpu.async_copy(
                bufs[c % nbuf],
                out_hbm.at[pl.ds(base + c * chunk, chunk), pl.ds(0, pdim)],
                ws[c % nbuf])
        for c in range(max(0, nch - nbuf), nch):
            if c in wh:
                wh[c].wait()

    return gk(table_pad, idx_flat)


def _stitch_body(tail_ref, idx_ref, prev_ref, out_ref, idx2_ref):
    del prev_ref
    out_ref[...] = tail_ref[...]
    kk = idx2_ref.shape[1]
    for r in range(_BLK // kk):
        idx2_ref[r, :] = idx_ref[pl.ds(r * kk, kk)]


def _stitch_tails(qmain, tails, idx_flat, mdim, bsz, k):
    nrows, dim = qmain.shape
    nb = nrows // _BLK
    cblk = mdim // 128
    rpb = _BLK // k
    return pl.pallas_call(
        _stitch_body,
        grid=(nb,),
        in_specs=[
            pl.BlockSpec((_BLK, 128), lambda i: (i, 0)),
            pl.BlockSpec((_BLK,), lambda i: (i,)),
            pl.BlockSpec(memory_space=pltpu.MemorySpace.HBM),
        ],
        out_specs=[
            pl.BlockSpec((_BLK, 128), lambda i, c=cblk: (i, c)),
            pl.BlockSpec((rpb, k), lambda i: (i, 0)),
        ],
        out_shape=[
            jax.ShapeDtypeStruct((nrows, dim), jnp.float32),
            jax.ShapeDtypeStruct((bsz, k), jnp.int32),
        ],
        input_output_aliases={2: 0},
    )(tails, idx_flat, qmain)


def kernel(slot_features, W, b, basis_vectors):
    bsz, k, d = slot_features.shape
    slots2d = slot_features.reshape(bsz * k, d)
    idx_flat, indices, loss, ent, tp = _vq_tc(slots2d, W, b.reshape(1, -1),
                                              basis_vectors, bsz, k)
    table_pad = tp.reshape(_NUM_CODES, _PDIM)
    q = _sc_gather_call(table_pad, idx_flat, _BASIS_DIM)
    return (q.reshape(bsz, k, _BASIS_DIM), indices,
            loss[0, 0], ent[0, 0])


# speedup vs baseline: 1.0329x; 1.0329x over previous
import functools

import jax
import jax.numpy as jnp
from jax import lax
from jax.experimental import pallas as pl
from jax.experimental.pallas import tpu as pltpu
from jax.experimental.pallas import tpu_sc as plsc

_NUM_CODES = 1024
_BASIS_DIM = 2700
_BETA = 0.25
_BLK = 256


_PDIM = (_BASIS_DIM + 127) // 128 * 128


def _vq_tc_body(slots_ref, wt_ref, b_ref, basist_ref,
                idx_ref, idx2t_ref, loss_ref, ent_ref,
                acc_ref, cn_ref, msum_ref):
    i = pl.program_id(0)
    nblk = pl.num_programs(0)

    @pl.when(i == 0)
    def _init():
        bsq = basist_ref[...] * basist_ref[...]
        cn_ref[...] = jnp.sum(bsq, axis=0, keepdims=True)
        acc_ref[...] = jnp.zeros_like(acc_ref)
        msum_ref[0] = 0.0

    z = lax.dot_general(slots_ref[...], wt_ref[...], (((1,), (1,)), ((), ())),
                        preferred_element_type=jnp.float32) + b_ref[...]
    g = lax.dot_general(z, basist_ref[...], (((1,), (0,)), ((), ())),
                        preferred_element_type=jnp.float32)
    zn = jnp.sum(z * z, axis=1, keepdims=True)
    dist = zn - 2.0 * g + cn_ref[...]
    m = jnp.min(dist, axis=1, keepdims=True)
    idx = jnp.argmin(dist, axis=1).astype(jnp.int32)
    idx_ref[...] = idx
    kk = idx2t_ref.shape[1]
    for r in range(_BLK // kk):
        idx2t_ref[r, :] = idx[r * kk:(r + 1) * kk]
    p = jnp.exp(m - dist)
    p = p / jnp.sum(p, axis=1, keepdims=True)
    acc_ref[...] += jnp.sum(p, axis=0, keepdims=True)
    msum_ref[0] += jnp.sum(m)

    @pl.when(i == nblk - 1)
    def _fin():
        nrows = nblk * _BLK
        avg = acc_ref[...] / nrows
        ent_ref[0, 0] = -jnp.sum(avg * jnp.log(avg + 1e-8))
        loss_ref[0, 0] = (_BETA / (nrows * _BASIS_DIM)) * msum_ref[0]


def _vq_tc(slots2d, wt, b2, basist, bsz, k):
    nrows, d = slots2d.shape
    nblk = nrows // _BLK
    rpb = _BLK // k
    return pl.pallas_call(
        _vq_tc_body,
        grid=(nblk,),
        in_specs=[
            pl.BlockSpec((_BLK, d), lambda i: (i, 0)),
            pl.BlockSpec(wt.shape, lambda i: (0, 0)),
            pl.BlockSpec(b2.shape, lambda i: (0, 0)),
            pl.BlockSpec(basist.shape, lambda i: (0, 0)),
        ],
        out_specs=[
            pl.BlockSpec((_BLK,), lambda i: (i,)),
            pl.BlockSpec((_BLK // k, k), lambda i: (i, 0)),
            pl.BlockSpec(memory_space=pltpu.SMEM),
            pl.BlockSpec(memory_space=pltpu.SMEM),
        ],
        out_shape=[
            jax.ShapeDtypeStruct((nblk * _BLK,), jnp.int32),
            jax.ShapeDtypeStruct((bsz, k), jnp.int32),
            jax.ShapeDtypeStruct((1, 1), jnp.float32),
            jax.ShapeDtypeStruct((1, 1), jnp.float32),
        ],
        scratch_shapes=[
            pltpu.VMEM((1, _NUM_CODES), jnp.float32),
            pltpu.VMEM((1, _NUM_CODES), jnp.float32),
            pltpu.SMEM((1,), jnp.float32),
        ],
    )(slots2d, wt, b2, basist)


def _sc_gather_call(table, idx_flat, dim):
    nrows = idx_flat.shape[0]
    pdim = _PDIM
    info = plsc.get_sparse_core_info()
    ncores = info.num_cores
    nw = ncores * info.num_subcores
    rpw = nrows // nw
    chunk = 8
    nbuf = 4
    nch = rpw // chunk

    @functools.partial(
        pl.kernel,
        out_type=jax.ShapeDtypeStruct((nrows, dim), jnp.float32),
        mesh=plsc.VectorSubcoreMesh(core_axis_name="c", subcore_axis_name="s"),
        scratch_types=(
            [pltpu.VMEM((rpw,), jnp.int32)]
            + [pltpu.VMEM((chunk, pdim), jnp.float32)] * nbuf
            + [pltpu.SemaphoreType.DMA] * (2 * nbuf)
        ),
    )
    def gk(table_hbm, idx_hbm, out_hbm, idx_v, *bufsem):
        bufs = bufsem[:nbuf]
        gs = bufsem[nbuf:2 * nbuf]
        ws = bufsem[2 * nbuf:]
        wid = lax.axis_index("s") * ncores + lax.axis_index("c")
        base = wid * rpw
        pltpu.sync_copy(idx_hbm.at[pl.ds(base, rpw)], idx_v)
        gh = {}
        wh = {}
        tbl = table_hbm.at[:, pl.ds(0, pdim)]
        for c in range(min(nbuf, nch)):
            gh[c] = pltpu.async_copy(
                tbl.at[idx_v.at[pl.ds(c * chunk, chunk)]],
                bufs[c % nbuf], gs[c % nbuf])
        for c in range(nch):
            nx = c + 2
            if nbuf <= nx < nch:
                wh[nx - nbuf].wait()
                gh[nx] = pltpu.async_copy(
                    tbl.at[idx_v.at[pl.ds(nx * chunk, chunk)]],
                    bufs[nx % nbuf], gs[nx % nbuf])
            gh[c].wait()
            wh[c] = pltpu.async_copy(
                bufs[c % nbuf],
                out_hbm.at[pl.ds(base + c * chunk, chunk), pl.ds(0, pdim)],
                ws[c % nbuf])
        for c in range(max(0, nch - nbuf), nch):
            if c in wh:
                wh[c].wait()

    return gk(table, idx_flat)


def _stitch_body(tail_ref, idx_ref, prev_ref, out_ref, idx2_ref):
    del prev_ref
    out_ref[...] = tail_ref[...]
    kk = idx2_ref.shape[1]
    for r in range(_BLK // kk):
        idx2_ref[r, :] = idx_ref[pl.ds(r * kk, kk)]


def _stitch_tails(qmain, tails, idx_flat, mdim, bsz, k):
    nrows, dim = qmain.shape
    nb = nrows // _BLK
    cblk = mdim // 128
    rpb = _BLK // k
    return pl.pallas_call(
        _stitch_body,
        grid=(nb,),
        in_specs=[
            pl.BlockSpec((_BLK, 128), lambda i: (i, 0)),
            pl.BlockSpec((_BLK,), lambda i: (i,)),
            pl.BlockSpec(memory_space=pltpu.MemorySpace.HBM),
        ],
        out_specs=[
            pl.BlockSpec((_BLK, 128), lambda i, c=cblk: (i, c)),
            pl.BlockSpec((rpb, k), lambda i: (i, 0)),
        ],
        out_shape=[
            jax.ShapeDtypeStruct((nrows, dim), jnp.float32),
            jax.ShapeDtypeStruct((bsz, k), jnp.int32),
        ],
        input_output_aliases={2: 0},
    )(tails, idx_flat, qmain)


def kernel(slot_features, W, b, basis_vectors):
    bsz, k, d = slot_features.shape
    slots2d = slot_features.reshape(bsz * k, d)
    idx_flat, indices, loss, ent = _vq_tc(slots2d, W.T, b.reshape(1, -1),
                                          basis_vectors.T, bsz, k)
    q = _sc_gather_call(basis_vectors, idx_flat, _BASIS_DIM)
    return (q.reshape(bsz, k, _BASIS_DIM), indices,
            loss[0, 0], ent[0, 0])


# speedup vs baseline: 1.0750x; 1.0407x over previous
import functools

import jax
import jax.numpy as jnp
from jax import lax
from jax.experimental import pallas as pl
from jax.experimental.pallas import tpu as pltpu
from jax.experimental.pallas import tpu_sc as plsc

_NUM_CODES = 1024
_BASIS_DIM = 2700
_BETA = 0.25
_BLK = 512


_PDIM = (_BASIS_DIM + 127) // 128 * 128


def _vq_tc_body(slots_ref, wt_ref, b_ref, basist_ref,
                idx_ref, idx2t_ref, loss_ref, ent_ref,
                acc_ref, cn_ref, msum_ref):
    i = pl.program_id(0)
    nblk = pl.num_programs(0)

    @pl.when(i == 0)
    def _init():
        bsq = basist_ref[...] * basist_ref[...]
        cn_ref[...] = jnp.sum(bsq, axis=0, keepdims=True)
        acc_ref[...] = jnp.zeros_like(acc_ref)
        msum_ref[0] = 0.0

    z = lax.dot_general(slots_ref[...], wt_ref[...], (((1,), (1,)), ((), ())),
                        preferred_element_type=jnp.float32) + b_ref[...][None, :]
    g = lax.dot_general(z, basist_ref[...], (((1,), (0,)), ((), ())),
                        preferred_element_type=jnp.float32)
    zn = jnp.sum(z * z, axis=1, keepdims=True)
    dist = zn - 2.0 * g + cn_ref[...]
    m = jnp.min(dist, axis=1, keepdims=True)
    idx = jnp.argmin(dist, axis=1).astype(jnp.int32)
    idx_ref[...] = idx
    kk = idx2t_ref.shape[1]
    for r in range(_BLK // kk):
        idx2t_ref[r, :] = idx[r * kk:(r + 1) * kk]
    p = jnp.exp(m - dist)
    p = p / jnp.sum(p, axis=1, keepdims=True)
    acc_ref[...] += jnp.sum(p, axis=0, keepdims=True)
    msum_ref[0] += jnp.sum(m)

    @pl.when(i == nblk - 1)
    def _fin():
        nrows = nblk * _BLK
        avg = acc_ref[...] / nrows
        ent_ref[0, 0] = -jnp.sum(avg * jnp.log(avg + 1e-8))
        loss_ref[0, 0] = (_BETA / (nrows * _BASIS_DIM)) * msum_ref[0]


def _vq_tc(slots2d, wt, b2, basist, bsz, k):
    nrows, d = slots2d.shape
    nblk = nrows // _BLK
    rpb = _BLK // k
    return pl.pallas_call(
        _vq_tc_body,
        grid=(nblk,),
        in_specs=[
            pl.BlockSpec((_BLK, d), lambda i: (i, 0)),
            pl.BlockSpec(wt.shape, lambda i: (0, 0)),
            pl.BlockSpec(b2.shape, lambda i: (0,)),
            pl.BlockSpec(basist.shape, lambda i: (0, 0)),
        ],
        out_specs=[
            pl.BlockSpec((_BLK,), lambda i: (i,)),
            pl.BlockSpec((_BLK // k, k), lambda i: (i, 0)),
            pl.BlockSpec(memory_space=pltpu.SMEM),
            pl.BlockSpec(memory_space=pltpu.SMEM),
        ],
        out_shape=[
            jax.ShapeDtypeStruct((nblk * _BLK,), jnp.int32),
            jax.ShapeDtypeStruct((bsz, k), jnp.int32),
            jax.ShapeDtypeStruct((1, 1), jnp.float32),
            jax.ShapeDtypeStruct((1, 1), jnp.float32),
        ],
        scratch_shapes=[
            pltpu.VMEM((1, _NUM_CODES), jnp.float32),
            pltpu.VMEM((1, _NUM_CODES), jnp.float32),
            pltpu.SMEM((1,), jnp.float32),
        ],
    )(slots2d, wt, b2, basist)


def _sc_gather_call(table, idx_flat, dim):
    nrows = idx_flat.shape[0]
    pdim = _PDIM
    info = plsc.get_sparse_core_info()
    ncores = info.num_cores
    nw = ncores * info.num_subcores
    rpw = nrows // nw
    chunk = 8
    nbuf = 4
    nch = rpw // chunk

    @functools.partial(
        pl.kernel,
        out_type=jax.ShapeDtypeStruct((nrows, dim), jnp.float32),
        mesh=plsc.VectorSubcoreMesh(core_axis_name="c", subcore_axis_name="s"),
        scratch_types=(
            [pltpu.VMEM((rpw,), jnp.int32)]
            + [pltpu.VMEM((chunk, pdim), jnp.float32)] * nbuf
            + [pltpu.SemaphoreType.DMA] * (2 * nbuf)
        ),
    )
    def gk(table_hbm, idx_hbm, out_hbm, idx_v, *bufsem):
        bufs = bufsem[:nbuf]
        gs = bufsem[nbuf:2 * nbuf]
        ws = bufsem[2 * nbuf:]
        wid = lax.axis_index("s") * ncores + lax.axis_index("c")
        base = wid * rpw
        pltpu.sync_copy(idx_hbm.at[pl.ds(base, rpw)], idx_v)
        gh = {}
        wh = {}
        tbl = table_hbm.at[:, pl.ds(0, pdim)]
        for c in range(min(nbuf, nch)):
            gh[c] = pltpu.async_copy(
                tbl.at[idx_v.at[pl.ds(c * chunk, chunk)]],
                bufs[c % nbuf], gs[c % nbuf])
        for c in range(nch):
            nx = c + 2
            if nbuf <= nx < nch:
                wh[nx - nbuf].wait()
                gh[nx] = pltpu.async_copy(
                    tbl.at[idx_v.at[pl.ds(nx * chunk, chunk)]],
                    bufs[nx % nbuf], gs[nx % nbuf])
            gh[c].wait()
            wh[c] = pltpu.async_copy(
                bufs[c % nbuf],
                out_hbm.at[pl.ds(base + c * chunk, chunk), pl.ds(0, pdim)],
                ws[c % nbuf])
        for c in range(max(0, nch - nbuf), nch):
            if c in wh:
                wh[c].wait()

    return gk(table, idx_flat)


def _stitch_body(tail_ref, idx_ref, prev_ref, out_ref, idx2_ref):
    del prev_ref
    out_ref[...] = tail_ref[...]
    kk = idx2_ref.shape[1]
    for r in range(_BLK // kk):
        idx2_ref[r, :] = idx_ref[pl.ds(r * kk, kk)]


def _stitch_tails(qmain, tails, idx_flat, mdim, bsz, k):
    nrows, dim = qmain.shape
    nb = nrows // _BLK
    cblk = mdim // 128
    rpb = _BLK // k
    return pl.pallas_call(
        _stitch_body,
        grid=(nb,),
        in_specs=[
            pl.BlockSpec((_BLK, 128), lambda i: (i, 0)),
            pl.BlockSpec((_BLK,), lambda i: (i,)),
            pl.BlockSpec(memory_space=pltpu.MemorySpace.HBM),
        ],
        out_specs=[
            pl.BlockSpec((_BLK, 128), lambda i, c=cblk: (i, c)),
            pl.BlockSpec((rpb, k), lambda i: (i, 0)),
        ],
        out_shape=[
            jax.ShapeDtypeStruct((nrows, dim), jnp.float32),
            jax.ShapeDtypeStruct((bsz, k), jnp.int32),
        ],
        input_output_aliases={2: 0},
    )(tails, idx_flat, qmain)


def kernel(slot_features, W, b, basis_vectors):
    bsz, k, d = slot_features.shape
    slots2d = slot_features.reshape(bsz * k, d)
    idx_flat, indices, loss, ent = _vq_tc(slots2d, W.T, b,
                                          basis_vectors.T, bsz, k)
    q = _sc_gather_call(basis_vectors, idx_flat, _BASIS_DIM)
    return (q.reshape(bsz, k, _BASIS_DIM), indices,
            loss[0, 0], ent[0, 0])
